# Initial kernel scaffold; baseline (speedup 1.0000x reference)
import jax
import jax.numpy as jnp
from jax.experimental import pallas as pl


def kernel(x, table):
    raise NotImplementedError("write your pallas kernel here")


# speedup vs baseline: 3.9659x; 3.9659x over previous
import functools

import jax
import jax.numpy as jnp
from jax import lax
from jax.experimental import pallas as pl
from jax.experimental.pallas import tpu as pltpu
from jax.experimental.pallas import tpu_sc as plsc

_D = 64
_B = 4096 * 200


def _build(B, D):
    info = plsc.get_sparse_core_info()
    NC, NS = info.num_cores, info.num_subcores
    NW = NC * NS
    b_per_w = B // NW
    CHUNK = 512
    NCHUNK = b_per_w // CHUNK

    mesh = plsc.VectorSubcoreMesh(core_axis_name="c", subcore_axis_name="s")

    @functools.partial(
        pl.kernel,
        out_type=jax.ShapeDtypeStruct((B, D), jnp.float32),
        mesh=mesh,
        scratch_types=[
            pltpu.VMEM((CHUNK,), jnp.int32),
            pltpu.VMEM((CHUNK, D), jnp.float32),
            pltpu.SemaphoreType.DMA,
        ],
        compiler_params=pltpu.CompilerParams(use_tc_tiling_on_sc=False),
    )
    def gather_kernel(x_hbm, table_hbm, out_hbm, idx_v, rows_v, sem):
        wid = lax.axis_index("s") * NC + lax.axis_index("c")
        base = wid * b_per_w

        def body(i, carry):
            off = base + i * CHUNK
            pltpu.sync_copy(x_hbm.at[pl.ds(off, CHUNK)], idx_v)
            pltpu.async_copy(table_hbm.at[idx_v], rows_v, sem).wait()
            pltpu.sync_copy(rows_v, out_hbm.at[pl.ds(off, CHUNK)])
            return carry

        lax.fori_loop(0, NCHUNK, body, 0)

    return gather_kernel


_gather = _build(_B, _D)


@jax.jit
def kernel(x, table):
    flat = x.reshape(-1)
    out = _gather(flat, table)
    return out.reshape(x.shape[0], x.shape[1], _D)


# speedup vs baseline: 4.2608x; 1.0744x over previous
import functools

import jax
import jax.numpy as jnp
from jax import lax
from jax.experimental import pallas as pl
from jax.experimental.pallas import tpu as pltpu
from jax.experimental.pallas import tpu_sc as plsc

_D = 64
_B = 4096 * 200


def _build(B, D):
    info = plsc.get_sparse_core_info()
    NC, NS = info.num_cores, info.num_subcores
    NW = NC * NS
    b_per_w = B // NW
    CHUNK = 256
    NBUF = 4
    NCHUNK = b_per_w // CHUNK
    NOUT = NCHUNK // NBUF

    mesh = plsc.VectorSubcoreMesh(core_axis_name="c", subcore_axis_name="s")

    @functools.partial(
        pl.kernel,
        out_type=jax.ShapeDtypeStruct((B, D), jnp.float32),
        mesh=mesh,
        scratch_types=[
            pltpu.VMEM((b_per_w,), jnp.int32),
            pltpu.VMEM((NBUF, CHUNK, D), jnp.float32),
            pltpu.SemaphoreType.DMA((NBUF,)),
            pltpu.SemaphoreType.DMA((NBUF,)),
        ],
        compiler_params=pltpu.CompilerParams(use_tc_tiling_on_sc=False),
    )
    def gather_kernel(x_hbm, table_hbm, out_hbm, idx_v, rows_v, gsem, ssem):
        wid = lax.axis_index("s") * NC + lax.axis_index("c")
        base = wid * b_per_w

        pltpu.sync_copy(x_hbm.at[pl.ds(base, b_per_w)], idx_v)

        def gather_chunk(i, b):
            pltpu.async_copy(
                table_hbm.at[idx_v.at[pl.ds(i * CHUNK, CHUNK)]],
                rows_v.at[b],
                gsem.at[b],
            )

        def store_chunk(i, b):
            pltpu.async_copy(
                rows_v.at[b],
                out_hbm.at[pl.ds(base + i * CHUNK, CHUNK)],
                ssem.at[b],
            )

        for b in range(NBUF):
            gather_chunk(b, b)

        def outer(o, carry):
            for b in range(NBUF):
                i = o * NBUF + b
                pltpu.make_async_copy(
                    table_hbm.at[idx_v.at[pl.ds(0, CHUNK)]],
                    rows_v.at[b],
                    gsem.at[b],
                ).wait()
                store_chunk(i, b)

                @pl.when(o < NOUT - 1)
                def _():
                    pltpu.make_async_copy(
                        rows_v.at[b],
                        out_hbm.at[pl.ds(base, CHUNK)],
                        ssem.at[b],
                    ).wait()
                    gather_chunk(i + NBUF, b)

            return carry

        lax.fori_loop(0, NOUT, outer, 0)

        for b in range(NBUF):
            pltpu.make_async_copy(
                rows_v.at[b],
                out_hbm.at[pl.ds(base, CHUNK)],
                ssem.at[b],
            ).wait()

    return gather_kernel


_gather = _build(_B, _D)


@jax.jit
def kernel(x, table):
    flat = x.reshape(-1)
    out = _gather(flat, table)
    return out.reshape(x.shape[0], x.shape[1], _D)
